# Initial kernel scaffold; baseline (speedup 1.0000x reference)
import functools

import jax
import jax.numpy as jnp
from jax import lax
from jax.experimental import pallas as pl
from jax.experimental.pallas import tpu as pltpu
from jax.experimental.pallas import tpu_sc as plsc

N_ATOMS = 1048576
TABLE_PAD = 128
LANES = 16

_info = plsc.get_sparse_core_info()
_NC = _info.num_cores
_NS = _info.num_subcores
NW = _NC * _NS
CHUNK = N_ATOMS // NW


def _body(x_hbm, z_hbm, scale_hbm, shift_hbm, out_hbm,
          scale_v, shift_v, x_v, z_v, o_v, sem):
    wid = lax.axis_index("s") * _NC + lax.axis_index("c")
    base = wid * CHUNK

    pltpu.sync_copy(scale_hbm, scale_v)
    pltpu.sync_copy(shift_hbm, shift_v)
    cx = pltpu.async_copy(x_hbm.at[pl.ds(base, CHUNK)], x_v, sem)
    cz = pltpu.async_copy(z_hbm.at[pl.ds(base, CHUNK)], z_v, sem)
    cx.wait()
    cz.wait()

    def step(i, carry):
        off = i * LANES
        idx = z_v[pl.ds(off, LANES)]
        s = plsc.load_gather(scale_v, [idx])
        b = plsc.load_gather(shift_v, [idx])
        o_v[pl.ds(off, LANES)] = s * x_v[pl.ds(off, LANES)] + b
        return carry

    lax.fori_loop(0, CHUNK // LANES, step, 0)

    pltpu.sync_copy(o_v, out_hbm.at[pl.ds(base, CHUNK)])


@jax.jit
def _run(x_flat, z_i32, scale_pad, shift_pad):
    k = functools.partial(
        pl.kernel,
        mesh=plsc.VectorSubcoreMesh(core_axis_name="c", subcore_axis_name="s"),
        out_type=jax.ShapeDtypeStruct((N_ATOMS,), jnp.float32),
        scratch_types=[
            pltpu.VMEM((TABLE_PAD,), jnp.float32),
            pltpu.VMEM((TABLE_PAD,), jnp.float32),
            pltpu.VMEM((CHUNK,), jnp.float32),
            pltpu.VMEM((CHUNK,), jnp.int32),
            pltpu.VMEM((CHUNK,), jnp.float32),
            pltpu.SemaphoreType.DMA,
        ],
    )(_body)
    return k(x_flat, z_i32, scale_pad, shift_pad)


def kernel(x, Z, scale, shift):
    n_species = scale.shape[0]
    x_flat = x.reshape(N_ATOMS)
    z_i32 = Z.astype(jnp.int32)
    scale_pad = jnp.zeros((TABLE_PAD,), jnp.float32).at[:n_species].set(
        scale.reshape(n_species))
    shift_pad = jnp.zeros((TABLE_PAD,), jnp.float32).at[:n_species].set(
        shift.reshape(n_species))
    out = _run(x_flat, z_i32, scale_pad, shift_pad)
    return out.reshape(N_ATOMS, 1)


# speedup vs baseline: 597.2645x; 597.2645x over previous
import functools

import jax
import jax.numpy as jnp
from jax import lax
from jax.experimental import pallas as pl
from jax.experimental.pallas import tpu as pltpu
from jax.experimental.pallas import tpu_sc as plsc

N_ATOMS = 1048576
TABLE_PAD = 128
LANES = 16

_info = plsc.get_sparse_core_info()
_NC = _info.num_cores
_NS = _info.num_subcores
NW = _NC * _NS
CHUNK = N_ATOMS // NW


def _body(x_hbm, z_hbm, scale_hbm, shift_hbm, out_hbm,
          scale_v, shift_v, x_v, z_v, o_v, sem):
    wid = lax.axis_index("s") * _NC + lax.axis_index("c")
    base = wid * CHUNK

    pltpu.sync_copy(scale_hbm, scale_v)
    pltpu.sync_copy(shift_hbm, shift_v)
    cx = pltpu.async_copy(x_hbm.at[pl.ds(base, CHUNK)], x_v, sem)
    cz = pltpu.async_copy(z_hbm.at[pl.ds(base, CHUNK)], z_v, sem)
    cx.wait()
    cz.wait()

    def step(i, carry):
        off = i * LANES
        idx = z_v[pl.ds(off, LANES)]
        s = plsc.load_gather(scale_v, [idx])
        b = plsc.load_gather(shift_v, [idx])
        o_v[pl.ds(off, LANES)] = s * x_v[pl.ds(off, LANES)] + b
        return carry

    lax.fori_loop(0, CHUNK // LANES, step, 0)

    pltpu.sync_copy(o_v, out_hbm.at[pl.ds(base, CHUNK)])


@jax.jit
def _run(x_flat, z_i32, scale_pad, shift_pad):
    k = functools.partial(
        pl.kernel,
        mesh=plsc.VectorSubcoreMesh(core_axis_name="c", subcore_axis_name="s"),
        out_type=jax.ShapeDtypeStruct((N_ATOMS,), jnp.float32),
        compiler_params=pltpu.CompilerParams(needs_layout_passes=False),
        scratch_types=[
            pltpu.VMEM((TABLE_PAD,), jnp.float32),
            pltpu.VMEM((TABLE_PAD,), jnp.float32),
            pltpu.VMEM((CHUNK,), jnp.float32),
            pltpu.VMEM((CHUNK,), jnp.int32),
            pltpu.VMEM((CHUNK,), jnp.float32),
            pltpu.SemaphoreType.DMA,
        ],
    )(_body)
    return k(x_flat, z_i32, scale_pad, shift_pad)


def kernel(x, Z, scale, shift):
    n_species = scale.shape[0]
    x_flat = x.reshape(N_ATOMS)
    z_i32 = Z.astype(jnp.int32)
    scale_pad = jnp.zeros((TABLE_PAD,), jnp.float32).at[:n_species].set(
        scale.reshape(n_species))
    shift_pad = jnp.zeros((TABLE_PAD,), jnp.float32).at[:n_species].set(
        shift.reshape(n_species))
    out = _run(x_flat, z_i32, scale_pad, shift_pad)
    return out.reshape(N_ATOMS, 1)


# speedup vs baseline: 773.4444x; 1.2950x over previous
import functools

import jax
import jax.numpy as jnp
from jax import lax
from jax.experimental import pallas as pl
from jax.experimental.pallas import tpu as pltpu
from jax.experimental.pallas import tpu_sc as plsc

N_ATOMS = 1048576
TABLE_PAD = 128
LANES = 16

_info = plsc.get_sparse_core_info()
_NC = _info.num_cores
_NS = _info.num_subcores
NW = _NC * _NS
CHUNK = N_ATOMS // NW


def _body(x_hbm, z_hbm, scale_hbm, shift_hbm, out_hbm,
          scale_v, shift_v, x_v, z_v, o_v, sem):
    wid = lax.axis_index("s") * _NC + lax.axis_index("c")
    base = wid * CHUNK

    pltpu.sync_copy(scale_hbm, scale_v)
    pltpu.sync_copy(shift_hbm, shift_v)
    cx = pltpu.async_copy(x_hbm.at[pl.ds(base, CHUNK)], x_v, sem)
    cz = pltpu.async_copy(z_hbm.at[pl.ds(base, CHUNK)], z_v, sem)
    cx.wait()
    cz.wait()

    @plsc.parallel_loop(0, CHUNK, step=LANES, unroll=8)
    def _loop(off):
        idx = z_v[pl.ds(off, LANES)]
        s = plsc.load_gather(scale_v, [idx])
        b = plsc.load_gather(shift_v, [idx])
        o_v[pl.ds(off, LANES)] = s * x_v[pl.ds(off, LANES)] + b

    pltpu.sync_copy(o_v, out_hbm.at[pl.ds(base, CHUNK)])


@jax.jit
def _run(x_flat, z_i32, scale_pad, shift_pad):
    k = functools.partial(
        pl.kernel,
        mesh=plsc.VectorSubcoreMesh(core_axis_name="c", subcore_axis_name="s"),
        out_type=jax.ShapeDtypeStruct((N_ATOMS,), jnp.float32),
        compiler_params=pltpu.CompilerParams(needs_layout_passes=False),
        scratch_types=[
            pltpu.VMEM((TABLE_PAD,), jnp.float32),
            pltpu.VMEM((TABLE_PAD,), jnp.float32),
            pltpu.VMEM((CHUNK,), jnp.float32),
            pltpu.VMEM((CHUNK,), jnp.int32),
            pltpu.VMEM((CHUNK,), jnp.float32),
            pltpu.SemaphoreType.DMA,
        ],
    )(_body)
    return k(x_flat, z_i32, scale_pad, shift_pad)


def kernel(x, Z, scale, shift):
    n_species = scale.shape[0]
    x_flat = x.reshape(N_ATOMS)
    z_i32 = Z.astype(jnp.int32)
    scale_pad = jnp.zeros((TABLE_PAD,), jnp.float32).at[:n_species].set(
        scale.reshape(n_species))
    shift_pad = jnp.zeros((TABLE_PAD,), jnp.float32).at[:n_species].set(
        shift.reshape(n_species))
    out = _run(x_flat, z_i32, scale_pad, shift_pad)
    return out.reshape(N_ATOMS, 1)


# speedup vs baseline: 821.9755x; 1.0627x over previous
import functools

import jax
import jax.numpy as jnp
from jax import lax
from jax.experimental import pallas as pl
from jax.experimental.pallas import tpu as pltpu
from jax.experimental.pallas import tpu_sc as plsc

N_ATOMS = 1048576
N_SP = 119
LANES = 16

_info = plsc.get_sparse_core_info()
_NC = _info.num_cores
_NS = _info.num_subcores
NW = _NC * _NS
CHUNK = N_ATOMS // NW
NSUB = 4
SUB = CHUNK // NSUB


def _body(x_hbm, z_hbm, scale_hbm, shift_hbm, out_hbm,
          scale_v, shift_v, x_v, z_v, o_v,
          sem_t, sem_in, sem_out):
    wid = lax.axis_index("s") * _NC + lax.axis_index("c")
    base = wid * CHUNK

    ct_s = pltpu.async_copy(scale_hbm, scale_v, sem_t)
    ct_b = pltpu.async_copy(shift_hbm, shift_v, sem_t)

    ins = []
    for s in range(NSUB):
        lo = base + s * SUB
        cz = pltpu.async_copy(z_hbm.at[pl.ds(lo, SUB)],
                              z_v.at[pl.ds(s * SUB, SUB)], sem_in[s])
        cx = pltpu.async_copy(x_hbm.at[pl.ds(lo, SUB)],
                              x_v.at[pl.ds(s * SUB, SUB)], sem_in[s])
        ins.append((cz, cx))

    ct_s.wait()
    ct_b.wait()

    outs = []
    for s in range(NSUB):
        cz, cx = ins[s]
        cz.wait()
        cx.wait()

        @plsc.parallel_loop(s * SUB, (s + 1) * SUB, step=LANES, unroll=8)
        def _loop(off):
            idx = z_v[pl.ds(off, LANES)]
            sc = plsc.load_gather(scale_v, [idx])
            sh = plsc.load_gather(shift_v, [idx])
            o_v[pl.ds(off, LANES)] = sc * x_v[pl.ds(off, LANES)] + sh

        co = pltpu.async_copy(o_v.at[pl.ds(s * SUB, SUB)],
                              out_hbm.at[pl.ds(base + s * SUB, SUB)],
                              sem_out)
        outs.append(co)

    for co in outs:
        co.wait()


@jax.jit
def _run(x_flat, z_i32, scale_flat, shift_flat):
    k = functools.partial(
        pl.kernel,
        mesh=plsc.VectorSubcoreMesh(core_axis_name="c", subcore_axis_name="s"),
        out_type=jax.ShapeDtypeStruct((N_ATOMS,), jnp.float32),
        compiler_params=pltpu.CompilerParams(needs_layout_passes=False),
        scratch_types=[
            pltpu.VMEM((N_SP,), jnp.float32),
            pltpu.VMEM((N_SP,), jnp.float32),
            pltpu.VMEM((CHUNK,), jnp.float32),
            pltpu.VMEM((CHUNK,), jnp.int32),
            pltpu.VMEM((CHUNK,), jnp.float32),
            pltpu.SemaphoreType.DMA,
            [pltpu.SemaphoreType.DMA] * NSUB,
            pltpu.SemaphoreType.DMA,
        ],
    )(_body)
    return k(x_flat, z_i32, scale_flat, shift_flat)


def kernel(x, Z, scale, shift):
    x_flat = x.reshape(N_ATOMS)
    z_i32 = Z.astype(jnp.int32)
    out = _run(x_flat, z_i32, scale.reshape(N_SP), shift.reshape(N_SP))
    return out.reshape(N_ATOMS, 1)


# speedup vs baseline: 823.7460x; 1.0022x over previous
import functools

import jax
import jax.numpy as jnp
from jax import lax
from jax.experimental import pallas as pl
from jax.experimental.pallas import tpu as pltpu
from jax.experimental.pallas import tpu_sc as plsc

N_ATOMS = 1048576
N_SP = 119
LANES = 16

_info = plsc.get_sparse_core_info()
_NC = _info.num_cores
_NS = _info.num_subcores
NW = _NC * _NS
CHUNK = N_ATOMS // NW
NSUB = 8
SUB = CHUNK // NSUB


def _body(x_hbm, z_hbm, scale_hbm, shift_hbm, out_hbm,
          scale_v, shift_v, x_v, z_v, o_v,
          sem_t, sem_in, sem_out):
    wid = lax.axis_index("s") * _NC + lax.axis_index("c")
    base = wid * CHUNK

    ct_s = pltpu.async_copy(scale_hbm, scale_v, sem_t)
    ct_b = pltpu.async_copy(shift_hbm, shift_v, sem_t)

    ins = []
    for s in range(NSUB):
        lo = base + s * SUB
        cz = pltpu.async_copy(z_hbm.at[pl.ds(lo, SUB)],
                              z_v.at[pl.ds(s * SUB, SUB)], sem_in[s])
        cx = pltpu.async_copy(x_hbm.at[pl.ds(lo, SUB)],
                              x_v.at[pl.ds(s * SUB, SUB)], sem_in[s])
        ins.append((cz, cx))

    ct_s.wait()
    ct_b.wait()

    outs = []
    for s in range(NSUB):
        cz, cx = ins[s]
        cz.wait()
        cx.wait()

        @plsc.parallel_loop(s * SUB, (s + 1) * SUB, step=LANES, unroll=8)
        def _loop(off):
            idx = z_v[pl.ds(off, LANES)]
            sc = plsc.load_gather(scale_v, [idx])
            sh = plsc.load_gather(shift_v, [idx])
            o_v[pl.ds(off, LANES)] = sc * x_v[pl.ds(off, LANES)] + sh

        co = pltpu.async_copy(o_v.at[pl.ds(s * SUB, SUB)],
                              out_hbm.at[pl.ds(base + s * SUB, SUB)],
                              sem_out)
        outs.append(co)

    for co in outs:
        co.wait()


@jax.jit
def _run(x_flat, z_i32, scale_flat, shift_flat):
    k = functools.partial(
        pl.kernel,
        mesh=plsc.VectorSubcoreMesh(core_axis_name="c", subcore_axis_name="s"),
        out_type=jax.ShapeDtypeStruct((N_ATOMS,), jnp.float32),
        compiler_params=pltpu.CompilerParams(needs_layout_passes=False),
        scratch_types=[
            pltpu.VMEM((N_SP,), jnp.float32),
            pltpu.VMEM((N_SP,), jnp.float32),
            pltpu.VMEM((CHUNK,), jnp.float32),
            pltpu.VMEM((CHUNK,), jnp.int32),
            pltpu.VMEM((CHUNK,), jnp.float32),
            pltpu.SemaphoreType.DMA,
            [pltpu.SemaphoreType.DMA] * NSUB,
            pltpu.SemaphoreType.DMA,
        ],
    )(_body)
    return k(x_flat, z_i32, scale_flat, shift_flat)


def kernel(x, Z, scale, shift):
    x_flat = x.reshape(N_ATOMS)
    z_i32 = Z.astype(jnp.int32)
    out = _run(x_flat, z_i32, scale.reshape(N_SP), shift.reshape(N_SP))
    return out.reshape(N_ATOMS, 1)
